# baseline (speedup 1.0000x reference)
import functools

import jax
import jax.numpy as jnp
from jax import lax
from jax.experimental import pallas as pl
from jax.experimental.pallas import tpu as pltpu
from jax.experimental.pallas import tpu_sc as plsc

BATCH = 16384
_VPAD = 1000448

try:
    _INFO = plsc.get_sparse_core_info()
    _NC = _INFO.num_cores
    _NS = _INFO.num_subcores
    _L = _INFO.num_lanes
except Exception:
    _NC, _NS, _L = 2, 16, 16
_NW = _NC * _NS
_BPW = BATCH // _NW
_CHUNK = 128
_NCHUNK = _BPW // _CHUNK
_NSL = _BPW // _L

_mesh = plsc.VectorSubcoreMesh(core_axis_name="c", subcore_axis_name="s")
_params = pltpu.CompilerParams(needs_layout_passes=False)


@functools.partial(
    pl.kernel,
    out_type=jax.ShapeDtypeStruct((BATCH,), jnp.float32),
    mesh=_mesh,
    compiler_params=_params,
    scratch_types=[
        pltpu.VMEM((_BPW,), jnp.int32),
        pltpu.VMEM((_NCHUNK, _CHUNK), jnp.int32),
        pltpu.VMEM((_BPW,), jnp.float32),
        pltpu.SemaphoreType.DMA,
    ],
)
def _gather_one(ids_hbm, tab_hbm, out_hbm, ids_v, idxc_v, rows_v, sem):
    wid = lax.axis_index("s") * _NC + lax.axis_index("c")
    base = wid * _BPW

    pltpu.sync_copy(ids_hbm.at[pl.ds(base, _BPW)], ids_v)
    for j in range(_NSL):
        idxc_v[j // 8, pl.ds((j % 8) * _L, _L)] = ids_v[pl.ds(j * _L, _L)]

    copies = []
    for j in range(_NCHUNK):
        sl = pl.ds(j * _CHUNK, _CHUNK)
        copies.append(pltpu.async_copy(tab_hbm.at[idxc_v.at[j]], rows_v.at[sl], sem))
    for c in copies:
        c.wait()

    pltpu.sync_copy(rows_v, out_hbm.at[pl.ds(base, _BPW)])


@functools.partial(
    pl.kernel,
    out_type=jax.ShapeDtypeStruct((BATCH,), jnp.float32),
    mesh=_mesh,
    compiler_params=_params,
    scratch_types=[
        pltpu.VMEM((_BPW,), jnp.float32),
        pltpu.VMEM((_BPW,), jnp.float32),
        pltpu.VMEM((_BPW,), jnp.float32),
        pltpu.VMEM((_BPW,), jnp.float32),
        pltpu.VMEM((_L,), jnp.float32),
        pltpu.VMEM((_BPW,), jnp.float32),
    ],
)
def _combine(ub_hbm, ib_hbm, ue_hbm, ie_hbm, w0_hbm, out_hbm,
             ub_v, ib_v, ue_v, ie_v, w0_v, out_v):
    wid = lax.axis_index("s") * _NC + lax.axis_index("c")
    base = wid * _BPW
    sl_all = pl.ds(base, _BPW)

    pltpu.sync_copy(ub_hbm.at[sl_all], ub_v)
    pltpu.sync_copy(ib_hbm.at[sl_all], ib_v)
    pltpu.sync_copy(ue_hbm.at[sl_all], ue_v)
    pltpu.sync_copy(ie_hbm.at[sl_all], ie_v)
    pltpu.sync_copy(w0_hbm, w0_v)

    w0r = w0_v[...]
    for j in range(_NSL):
        sl = pl.ds(j * _L, _L)
        out_v[sl] = w0r + ub_v[sl] + ib_v[sl] + ue_v[sl] * ie_v[sl]

    pltpu.sync_copy(out_v, out_hbm.at[pl.ds(base, _BPW)])


def _flat(table):
    n = table.shape[0]
    return jnp.pad(table, ((0, _VPAD - n), (0, 0))).reshape(-1)


def kernel(INPUT, userBias, itemBias, userEmbed, itemEmbed, w0):
    ids = INPUT.astype(jnp.int32)
    u_ids = ids[:, 0]
    i_ids = ids[:, 1]
    ub = _gather_one(u_ids, _flat(userBias))
    ib = _gather_one(i_ids, _flat(itemBias))
    ue = _gather_one(u_ids, _flat(userEmbed))
    ie = _gather_one(i_ids, _flat(itemEmbed))
    out = _combine(ub, ib, ue, ie, jnp.broadcast_to(w0.reshape(()), (_L,)))
    return out.reshape(BATCH, 1)


# speedup vs baseline: 1.1586x; 1.1586x over previous
import functools

import jax
import jax.numpy as jnp
from jax import lax
from jax.experimental import pallas as pl
from jax.experimental.pallas import tpu as pltpu
from jax.experimental.pallas import tpu_sc as plsc

BATCH = 16384
_VPAD = 1000448

try:
    _INFO = plsc.get_sparse_core_info()
    _NC = _INFO.num_cores
    _NS = _INFO.num_subcores
    _L = _INFO.num_lanes
except Exception:
    _NC, _NS, _L = 2, 16, 16
_NW = _NC * _NS
_BPW = BATCH // _NW
_CHUNK = 128
_NCHUNK = _BPW // _CHUNK
_NSL = _BPW // _L

_mesh = plsc.VectorSubcoreMesh(core_axis_name="c", subcore_axis_name="s")


@functools.partial(
    pl.kernel,
    out_type=jax.ShapeDtypeStruct((BATCH,), jnp.float32),
    mesh=_mesh,
    compiler_params=pltpu.CompilerParams(needs_layout_passes=False),
    scratch_types=[
        pltpu.VMEM((_BPW,), jnp.int32),
        pltpu.VMEM((_BPW,), jnp.int32),
        pltpu.VMEM((_NCHUNK, _CHUNK), jnp.int32),
        pltpu.VMEM((_NCHUNK, _CHUNK), jnp.int32),
        pltpu.VMEM((_BPW,), jnp.float32),
        pltpu.VMEM((_BPW,), jnp.float32),
        pltpu.VMEM((_BPW,), jnp.float32),
        pltpu.VMEM((_BPW,), jnp.float32),
        pltpu.VMEM((_L,), jnp.float32),
        pltpu.VMEM((_BPW,), jnp.float32),
        pltpu.SemaphoreType.DMA,
    ],
)
def _fm_sc(u_hbm, i_hbm, ub_hbm, ib_hbm, ue_hbm, ie_hbm, w0_hbm, out_hbm,
           u_v, i_v, uidx_v, iidx_v, ub_v, ib_v, ue_v, ie_v, w0_v, out_v, sem):
    wid = lax.axis_index("s") * _NC + lax.axis_index("c")
    base = wid * _BPW

    pltpu.sync_copy(u_hbm.at[pl.ds(base, _BPW)], u_v)
    pltpu.sync_copy(i_hbm.at[pl.ds(base, _BPW)], i_v)
    pltpu.sync_copy(w0_hbm, w0_v)

    for j in range(_NSL):
        sl = pl.ds(j * _L, _L)
        uidx_v[j // 8, pl.ds((j % 8) * _L, _L)] = u_v[sl]
        iidx_v[j // 8, pl.ds((j % 8) * _L, _L)] = i_v[sl]

    copies = []
    for j in range(_NCHUNK):
        sl = pl.ds(j * _CHUNK, _CHUNK)
        copies.append(pltpu.async_copy(ub_hbm.at[uidx_v.at[j]], ub_v.at[sl], sem))
        copies.append(pltpu.async_copy(ib_hbm.at[iidx_v.at[j]], ib_v.at[sl], sem))
        copies.append(pltpu.async_copy(ue_hbm.at[uidx_v.at[j]], ue_v.at[sl], sem))
        copies.append(pltpu.async_copy(ie_hbm.at[iidx_v.at[j]], ie_v.at[sl], sem))
    for c in copies:
        c.wait()

    w0r = w0_v[...]
    for j in range(_NSL):
        sl = pl.ds(j * _L, _L)
        out_v[sl] = w0r + ub_v[sl] + ib_v[sl] + ue_v[sl] * ie_v[sl]

    pltpu.sync_copy(out_v, out_hbm.at[pl.ds(base, _BPW)])


def _flat(table):
    n = table.shape[0]
    return jnp.pad(table, ((0, _VPAD - n), (0, 0))).reshape(-1)


def kernel(INPUT, userBias, itemBias, userEmbed, itemEmbed, w0):
    ids = INPUT.astype(jnp.int32)
    out = _fm_sc(
        ids[:, 0],
        ids[:, 1],
        _flat(userBias),
        _flat(itemBias),
        _flat(userEmbed),
        _flat(itemEmbed),
        jnp.broadcast_to(w0.reshape(()), (_L,)),
    )
    return out.reshape(BATCH, 1)
